# baseline (speedup 1.0000x reference)
import functools

import jax
import jax.numpy as jnp
from jax import lax
from jax.experimental import pallas as pl
from jax.experimental.pallas import tpu as pltpu
from jax.experimental.pallas import tpu_sc as plsc

_D = 64
_GRP = 16


@functools.partial(jax.jit, static_argnums=(2, 3, 4))
def _embed_lookup_t(idx, table_flat, nc, ns, b_per_w):
    B = idx.shape[0]
    V = table_flat.shape[0] // _D
    mesh = plsc.VectorSubcoreMesh(core_axis_name="c", subcore_axis_name="s")

    @functools.partial(
        pl.kernel,
        mesh=mesh,
        out_type=jax.ShapeDtypeStruct((_D, B), jnp.float32),
        scratch_types=[
            pltpu.VMEM((V * _D,), jnp.float32),
            pltpu.VMEM((b_per_w,), jnp.int32),
            pltpu.VMEM((_D, b_per_w), jnp.float32),
        ],
        compiler_params=pltpu.CompilerParams(needs_layout_passes=False),
    )
    def body(table_hbm, idx_hbm, out_hbm, tab_v, idx_v, buf):
        wid = lax.axis_index("s") * nc + lax.axis_index("c")
        base = wid * b_per_w
        pltpu.sync_copy(table_hbm, tab_v)
        pltpu.sync_copy(idx_hbm.at[pl.ds(base, b_per_w)], idx_v)

        def step(g, carry):
            iv = idx_v[pl.ds(g * _GRP, _GRP)]
            fidx = iv * _D
            for j in range(_D):
                v = plsc.load_gather(tab_v, [fidx + j])
                buf[j, pl.ds(g * _GRP, _GRP)] = v
            return carry

        lax.fori_loop(0, b_per_w // _GRP, step, 0)
        pltpu.sync_copy(buf, out_hbm.at[:, pl.ds(base, b_per_w)])

    return body(table_flat, idx)


def kernel(feature_names, emb_weight):
    idx = feature_names.astype(jnp.int32)
    table_flat = emb_weight.reshape(-1)
    info = plsc.get_sparse_core_info()
    nc, ns = info.num_cores, info.num_subcores
    b_per_w = idx.shape[0] // (nc * ns)
    return _embed_lookup_t(idx, table_flat, nc, ns, b_per_w).T


# speedup vs baseline: 1.8331x; 1.8331x over previous
import functools

import jax
import jax.numpy as jnp
from jax import lax
from jax.experimental import pallas as pl
from jax.experimental.pallas import tpu as pltpu
from jax.experimental.pallas import tpu_sc as plsc

_D = 64
_VPAD = 128
_GRP = 16


@functools.partial(jax.jit, static_argnums=(2, 3, 4))
def _embed_lookup_t(idx, table_t, nc, ns, b_per_w):
    B = idx.shape[0]
    mesh = plsc.VectorSubcoreMesh(core_axis_name="c", subcore_axis_name="s")

    @functools.partial(
        pl.kernel,
        mesh=mesh,
        out_type=jax.ShapeDtypeStruct((_D, B), jnp.float32),
        scratch_types=[
            pltpu.VMEM((_D, _VPAD), jnp.float32),
            pltpu.VMEM((b_per_w,), jnp.int32),
            pltpu.VMEM((_D, b_per_w), jnp.float32),
            pltpu.SemaphoreType.DMA,
            pltpu.SemaphoreType.DMA,
        ],
        compiler_params=pltpu.CompilerParams(needs_layout_passes=False),
    )
    def body(table_hbm, idx_hbm, out_hbm, tab_v, idx_v, buf, ssem, osem):
        wid = lax.axis_index("s") * nc + lax.axis_index("c")
        base = wid * b_per_w
        c_tab = pltpu.async_copy(
            table_hbm.at[pl.ds(0, _D), pl.ds(0, _VPAD)], tab_v, ssem
        )
        c_idx = pltpu.async_copy(
            idx_hbm.at[pl.ds(base, b_per_w)], idx_v, ssem
        )
        c_tab.wait()
        c_idx.wait()

        def step(g, carry):
            iv = idx_v[pl.ds(g * _GRP, _GRP)]
            depth = 6

            def ld(j):
                jv = jnp.full((_GRP,), j, jnp.int32)
                return plsc.load_gather(tab_v, [jv, iv])

            pipe = [ld(k) for k in range(depth)]
            for j in range(_D):
                if j + depth < _D:
                    pipe.append(ld(j + depth))
                buf[j, pl.ds(g * _GRP, _GRP)] = pipe.pop(0)
            return carry

        n_grp = b_per_w // _GRP
        nq = 4
        gq = n_grp // nq
        qw = b_per_w // nq
        copies = []
        for q in range(nq):
            lax.fori_loop(q * gq, (q + 1) * gq, step, 0)
            copies.append(
                pltpu.async_copy(
                    buf.at[:, pl.ds(q * qw, qw)],
                    out_hbm.at[:, pl.ds(base + q * qw, qw)],
                    osem,
                )
            )
        for c in copies:
            c.wait()

    return body(table_t, idx)


def kernel(feature_names, emb_weight):
    idx = feature_names.astype(jnp.int32)
    info = plsc.get_sparse_core_info()
    nc, ns = info.num_cores, info.num_subcores
    b_per_w = idx.shape[0] // (nc * ns)
    return _embed_lookup_t(idx, emb_weight.T, nc, ns, b_per_w).T


# speedup vs baseline: 1.8528x; 1.0107x over previous
import functools

import jax
import jax.numpy as jnp
from jax import lax
from jax.experimental import pallas as pl
from jax.experimental.pallas import tpu as pltpu
from jax.experimental.pallas import tpu_sc as plsc

_D = 64
_VPAD = 128
_GRP = 16
_STRIDE = 65
_VCAP = 112


@functools.partial(jax.jit, static_argnums=(2, 3, 4))
def _embed_lookup_t(idx, table_t, nc, ns, b_per_w):
    B = idx.shape[0]
    mesh = plsc.VectorSubcoreMesh(core_axis_name="c", subcore_axis_name="s")

    @functools.partial(
        pl.kernel,
        mesh=mesh,
        out_type=jax.ShapeDtypeStruct((_D, B), jnp.float32),
        scratch_types=[
            pltpu.VMEM((_D, _VPAD), jnp.float32),
            pltpu.VMEM((_VCAP * _STRIDE,), jnp.float32),
            pltpu.VMEM((b_per_w,), jnp.int32),
            pltpu.VMEM((_D, b_per_w), jnp.float32),
            pltpu.SemaphoreType.DMA,
            pltpu.SemaphoreType.DMA,
        ],
        compiler_params=pltpu.CompilerParams(needs_layout_passes=False),
    )
    def body(table_hbm, idx_hbm, out_hbm, tab2_v, tab_v, idx_v, buf, ssem, osem):
        wid = lax.axis_index("s") * nc + lax.axis_index("c")
        base = wid * b_per_w
        c_tab = pltpu.async_copy(
            table_hbm.at[pl.ds(0, _D), pl.ds(0, _VPAD)], tab2_v, ssem
        )
        c_idx = pltpu.async_copy(
            idx_hbm.at[pl.ds(base, b_per_w)], idx_v, ssem
        )
        c_tab.wait()

        cvecs = [
            jnp.arange(c * _GRP, (c + 1) * _GRP, dtype=jnp.int32) * _STRIDE
            for c in range(_VCAP // _GRP)
        ]

        def restride(j, carry):
            for c in range(_VCAP // _GRP):
                v = tab2_v[j, pl.ds(c * _GRP, _GRP)]
                plsc.store_scatter(tab_v, [cvecs[c] + j], v)
            return carry

        lax.fori_loop(0, _D, restride, 0)
        c_idx.wait()

        def step(g, carry):
            iv = idx_v[pl.ds(g * _GRP, _GRP)]
            fidx = iv * _STRIDE
            depth = 6
            pipe = [plsc.load_gather(tab_v, [fidx + k]) for k in range(depth)]
            for j in range(_D):
                if j + depth < _D:
                    pipe.append(plsc.load_gather(tab_v, [fidx + (j + depth)]))
                buf[j, pl.ds(g * _GRP, _GRP)] = pipe.pop(0)
            return carry

        n_grp = b_per_w // _GRP
        nq = 4
        gq = n_grp // nq
        qw = b_per_w // nq
        copies = []
        for q in range(nq):
            lax.fori_loop(q * gq, (q + 1) * gq, step, 0)
            copies.append(
                pltpu.async_copy(
                    buf.at[:, pl.ds(q * qw, qw)],
                    out_hbm.at[:, pl.ds(base + q * qw, qw)],
                    osem,
                )
            )
        for c in copies:
            c.wait()

    return body(table_t, idx)


def kernel(feature_names, emb_weight):
    idx = feature_names.astype(jnp.int32)
    info = plsc.get_sparse_core_info()
    nc, ns = info.num_cores, info.num_subcores
    b_per_w = idx.shape[0] // (nc * ns)
    return _embed_lookup_t(idx, emb_weight.T, nc, ns, b_per_w).T


# speedup vs baseline: 1.9785x; 1.0679x over previous
import functools

import jax
import jax.numpy as jnp
from jax import lax
from jax.experimental import pallas as pl
from jax.experimental.pallas import tpu as pltpu
from jax.experimental.pallas import tpu_sc as plsc

_D = 64
_STRIDE = 65
_GRP = 16


@functools.partial(jax.jit, static_argnums=(2, 3, 4))
def _embed_lookup_t(idx, table_flat, nc, ns, b_per_w):
    B = idx.shape[0]
    V = table_flat.shape[0] // _STRIDE
    mesh = plsc.VectorSubcoreMesh(core_axis_name="c", subcore_axis_name="s")

    @functools.partial(
        pl.kernel,
        mesh=mesh,
        out_type=jax.ShapeDtypeStruct((_D, B), jnp.float32),
        scratch_types=[
            pltpu.VMEM((V * _STRIDE,), jnp.float32),
            pltpu.VMEM((b_per_w,), jnp.int32),
            pltpu.VMEM((_D, b_per_w), jnp.float32),
            pltpu.SemaphoreType.DMA,
        ],
        compiler_params=pltpu.CompilerParams(needs_layout_passes=False),
    )
    def body(table_hbm, idx_hbm, out_hbm, tab_v, idx_v, buf, osem):
        wid = lax.axis_index("s") * nc + lax.axis_index("c")
        base = wid * b_per_w
        pltpu.sync_copy(table_hbm, tab_v)
        pltpu.sync_copy(idx_hbm.at[pl.ds(base, b_per_w)], idx_v)

        def step(g, carry):
            iv = idx_v[pl.ds(g * _GRP, _GRP)]
            fidx = iv * _STRIDE
            depth = 6
            pipe = [plsc.load_gather(tab_v, [fidx + k]) for k in range(depth)]
            for j in range(_D):
                if j + depth < _D:
                    pipe.append(
                        plsc.load_gather(tab_v, [fidx + (j + depth)])
                    )
                buf[j, pl.ds(g * _GRP, _GRP)] = pipe.pop(0)
            return carry

        n_grp = b_per_w // _GRP
        half_w = b_per_w // 2
        lax.fori_loop(0, n_grp // 2, step, 0)
        c1 = pltpu.async_copy(
            buf.at[:, pl.ds(0, half_w)],
            out_hbm.at[:, pl.ds(base, half_w)],
            osem,
        )
        lax.fori_loop(n_grp // 2, n_grp, step, 0)
        c2 = pltpu.async_copy(
            buf.at[:, pl.ds(half_w, half_w)],
            out_hbm.at[:, pl.ds(base + half_w, half_w)],
            osem,
        )
        c1.wait()
        c2.wait()

    return body(table_flat, idx)


def kernel(feature_names, emb_weight):
    idx = feature_names.astype(jnp.int32)
    table_flat = jnp.pad(
        emb_weight, ((0, 0), (0, _STRIDE - _D))
    ).reshape(-1)
    info = plsc.get_sparse_core_info()
    nc, ns = info.num_cores, info.num_subcores
    b_per_w = idx.shape[0] // (nc * ns)
    return _embed_lookup_t(idx, table_flat, nc, ns, b_per_w).T


# speedup vs baseline: 1.9902x; 1.0059x over previous
import functools

import jax
import jax.numpy as jnp
from jax import lax
from jax.experimental import pallas as pl
from jax.experimental.pallas import tpu as pltpu
from jax.experimental.pallas import tpu_sc as plsc

_D = 64
_STRIDE = 65
_GRP = 16


@functools.partial(jax.jit, static_argnums=(2, 3, 4))
def _embed_lookup_t(idx, table_flat, nc, ns, b_per_w):
    B = idx.shape[0]
    V = table_flat.shape[0] // _STRIDE
    mesh = plsc.VectorSubcoreMesh(core_axis_name="c", subcore_axis_name="s")

    @functools.partial(
        pl.kernel,
        mesh=mesh,
        out_type=jax.ShapeDtypeStruct((_D, B), jnp.float32),
        scratch_types=[
            pltpu.VMEM((V * _STRIDE,), jnp.float32),
            pltpu.VMEM((b_per_w,), jnp.int32),
            pltpu.VMEM((_D, b_per_w), jnp.float32),
            pltpu.SemaphoreType.DMA,
        ],
        compiler_params=pltpu.CompilerParams(
            needs_layout_passes=False, skip_device_barrier=True
        ),
    )
    def body(table_hbm, idx_hbm, out_hbm, tab_v, idx_v, buf, osem):
        wid = lax.axis_index("s") * nc + lax.axis_index("c")
        base = wid * b_per_w
        pltpu.sync_copy(table_hbm, tab_v)
        pltpu.sync_copy(idx_hbm.at[pl.ds(base, b_per_w)], idx_v)

        def step(g, carry):
            iv = idx_v[pl.ds(g * _GRP, _GRP)]
            fidx = iv * _STRIDE
            depth = 6
            pipe = [plsc.load_gather(tab_v, [fidx + k]) for k in range(depth)]
            for j in range(_D):
                if j + depth < _D:
                    pipe.append(
                        plsc.load_gather(tab_v, [fidx + (j + depth)])
                    )
                buf[j, pl.ds(g * _GRP, _GRP)] = pipe.pop(0)
            return carry

        n_grp = b_per_w // _GRP
        half_w = b_per_w // 2
        lax.fori_loop(0, n_grp // 2, step, 0)
        c1 = pltpu.async_copy(
            buf.at[:, pl.ds(0, half_w)],
            out_hbm.at[:, pl.ds(base, half_w)],
            osem,
        )
        lax.fori_loop(n_grp // 2, n_grp, step, 0)
        c2 = pltpu.async_copy(
            buf.at[:, pl.ds(half_w, half_w)],
            out_hbm.at[:, pl.ds(base + half_w, half_w)],
            osem,
        )
        c1.wait()
        c2.wait()

    return body(table_flat, idx)


def kernel(feature_names, emb_weight):
    idx = feature_names.astype(jnp.int32)
    table_flat = jnp.pad(
        emb_weight, ((0, 0), (0, _STRIDE - _D))
    ).reshape(-1)
    info = plsc.get_sparse_core_info()
    nc, ns = info.num_cores, info.num_subcores
    b_per_w = idx.shape[0] // (nc * ns)
    return _embed_lookup_t(idx, table_flat, nc, ns, b_per_w).T
